# baseline (speedup 1.0000x reference)
import jax
import jax.numpy as jnp
from jax.experimental import pallas as pl
from jax.experimental.pallas import tpu as pltpu

_NUM_DMAS = 16


def _copy_body(x_ref, o_ref, sems):
    rows = x_ref.shape[0] // _NUM_DMAS
    copies = [
        pltpu.make_async_copy(
            x_ref.at[pl.ds(k * rows, rows)],
            o_ref.at[pl.ds(k * rows, rows)],
            sems.at[k],
        )
        for k in range(_NUM_DMAS)
    ]
    for c in copies:
        c.start()
    for c in copies:
        c.wait()


def kernel(x):
    B, C, N = x.shape
    total_rows = B * C
    x2 = x.reshape(total_rows, N)
    out = pl.pallas_call(
        _copy_body,
        in_specs=[pl.BlockSpec(memory_space=pl.ANY)],
        out_specs=pl.BlockSpec(memory_space=pl.ANY),
        out_shape=jax.ShapeDtypeStruct((total_rows, N), x.dtype),
        scratch_shapes=[pltpu.SemaphoreType.DMA((_NUM_DMAS,))],
    )(x2)
    return out.reshape(B, C * 4, N // 4)


# speedup vs baseline: 18.2793x; 18.2793x over previous
import jax
import jax.numpy as jnp
from jax.experimental import pallas as pl
from jax.experimental.pallas import tpu as pltpu

_ROWS_PER_BLOCK = 256


def _copy_body(x_ref, o_ref):
    o_ref[...] = x_ref[...]


def kernel(x):
    B, C, N = x.shape
    total_rows = B * C
    x2 = x.reshape(total_rows, N)
    grid = total_rows // _ROWS_PER_BLOCK
    out = pl.pallas_call(
        _copy_body,
        grid=(grid,),
        in_specs=[pl.BlockSpec((_ROWS_PER_BLOCK, N), lambda i: (i, 0))],
        out_specs=pl.BlockSpec((_ROWS_PER_BLOCK, N), lambda i: (i, 0)),
        out_shape=jax.ShapeDtypeStruct((total_rows, N), x.dtype),
        compiler_params=pltpu.CompilerParams(
            dimension_semantics=("parallel",),
        ),
    )(x2)
    return out.reshape(B, C * 4, N // 4)


# speedup vs baseline: 18.4861x; 1.0113x over previous
import jax
import jax.numpy as jnp
from jax.experimental import pallas as pl
from jax.experimental.pallas import tpu as pltpu

_ROWS_PER_BLOCK = 512


def _copy_body(x_ref, o_ref):
    o_ref[...] = x_ref[...]


def kernel(x):
    B, C, N = x.shape
    total_rows = B * C
    x2 = x.reshape(total_rows, N)
    grid = total_rows // _ROWS_PER_BLOCK
    out = pl.pallas_call(
        _copy_body,
        grid=(grid,),
        in_specs=[pl.BlockSpec((_ROWS_PER_BLOCK, N), lambda i: (i, 0))],
        out_specs=pl.BlockSpec((_ROWS_PER_BLOCK, N), lambda i: (i, 0)),
        out_shape=jax.ShapeDtypeStruct((total_rows, N), x.dtype),
        compiler_params=pltpu.CompilerParams(
            dimension_semantics=("arbitrary",),
        ),
    )(x2)
    return out.reshape(B, C * 4, N // 4)
